# baseline (speedup 1.0000x reference)
import functools

import jax
import jax.numpy as jnp
from jax import lax
from jax.experimental import pallas as pl
from jax.experimental.pallas import tpu as pltpu
from jax.experimental.pallas import tpu_sc as plsc

NUM_CORES = 2
NUM_SUBCORES = 16
LANES = 16
NUM_WORKERS = NUM_CORES * NUM_SUBCORES

DMA_INFLIGHT = 8


def kernel(x, row_embed, col_embed):
    b_sz = x.shape[0]
    h, w = x.shape[-2], x.shape[-1]
    n_pos, c = row_embed.shape
    c2 = 2 * c
    ch_per_w = c2 // NUM_WORKERS

    mesh = plsc.VectorSubcoreMesh(
        core_axis_name="c",
        subcore_axis_name="s",
        num_cores=NUM_CORES,
        num_subcores=NUM_SUBCORES,
    )

    @functools.partial(
        pl.kernel,
        out_type=jax.ShapeDtypeStruct((b_sz, c2, h, w), jnp.float32),
        mesh=mesh,
        scratch_types=[
            pltpu.VMEM((n_pos, c), jnp.float32),
            pltpu.VMEM((ch_per_w, h, w), jnp.float32),
            pltpu.SemaphoreType.DMA,
        ],
        compiler_params=pltpu.CompilerParams(
            use_tc_tiling_on_sc=False, needs_layout_passes=False
        ),
    )
    def pos_kernel(row_hbm, col_hbm, out_hbm, table_v, chunk_v, sem):
        cid = lax.axis_index("c")
        sid = lax.axis_index("s")
        wid = sid * NUM_CORES + cid
        c0 = wid * ch_per_w
        is_col = c0 < c
        cbase = jnp.where(is_col, c0, c0 - c)

        @pl.when(is_col)
        def _():
            pltpu.sync_copy(col_hbm, table_v)

        @pl.when(jnp.logical_not(is_col))
        def _():
            pltpu.sync_copy(row_hbm, table_v)

        lane_iota = lax.iota(jnp.int32, LANES)

        @pl.when(is_col)
        def _():
            def k_body(k, _):
                ch = jnp.full((LANES,), cbase + k, jnp.int32)

                def jh_body(jh, _):
                    vec = plsc.load_gather(table_v, [jh * LANES + lane_iota, ch])

                    def i_body(i, _):
                        chunk_v[k, i, pl.ds(jh * LANES, LANES)] = vec
                        return 0

                    return lax.fori_loop(0, h, i_body, 0)

                lax.fori_loop(0, w // LANES, jh_body, 0)
                return 0

            lax.fori_loop(0, ch_per_w, k_body, 0)

        @pl.when(jnp.logical_not(is_col))
        def _():
            def k_body(k, _):
                ch = jnp.full((LANES,), cbase + k, jnp.int32)

                def i_body(i, _):
                    vec = plsc.load_gather(
                        table_v, [jnp.full((LANES,), i, jnp.int32), ch]
                    )

                    def jh_body(jh, _):
                        chunk_v[k, i, pl.ds(jh * LANES, LANES)] = vec
                        return 0

                    return lax.fori_loop(0, w // LANES, jh_body, 0)

                lax.fori_loop(0, h, i_body, 0)
                return 0

            lax.fori_loop(0, ch_per_w, k_body, 0)

        pending = []
        for b in range(b_sz):
            cp = pltpu.async_copy(chunk_v, out_hbm.at[b, pl.ds(c0, ch_per_w)], sem)
            pending.append(cp)
            if len(pending) >= DMA_INFLIGHT:
                pending.pop(0).wait()
        for cp in pending:
            cp.wait()

    return pos_kernel(row_embed, col_embed)


# speedup vs baseline: 8.5962x; 8.5962x over previous
import functools

import jax
import jax.numpy as jnp
from jax import lax
from jax.experimental import pallas as pl
from jax.experimental.pallas import tpu as pltpu
from jax.experimental.pallas import tpu_sc as plsc

NUM_CORES = 2
NUM_SUBCORES = 16
LANES = 16
NUM_WORKERS = NUM_CORES * NUM_SUBCORES

SUB = 8
LANE_TILE = 128

DMA_INFLIGHT = 8


def kernel(x, row_embed, col_embed):
    b_sz = x.shape[0]
    h, w = x.shape[-2], x.shape[-1]
    n_pos, c = row_embed.shape
    c2 = 2 * c
    wt = w // SUB
    ct = c2 // LANE_TILE
    ct_half = c // LANE_TILE

    n_egroups = 8
    n_bgroups = NUM_WORKERS // n_egroups
    i_per_w = h // n_egroups
    b_per_w = b_sz // n_bgroups

    mesh = plsc.VectorSubcoreMesh(
        core_axis_name="c",
        subcore_axis_name="s",
        num_cores=NUM_CORES,
        num_subcores=NUM_SUBCORES,
    )

    @functools.partial(
        pl.kernel,
        out_type=jax.ShapeDtypeStruct(
            (b_sz, h, wt, ct, SUB, LANE_TILE), jnp.float32
        ),
        mesh=mesh,
        scratch_types=[
            pltpu.VMEM((h, c), jnp.float32),
            pltpu.VMEM((h, c), jnp.float32),
            pltpu.VMEM((i_per_w, wt, ct, SUB, LANE_TILE), jnp.float32),
            pltpu.SemaphoreType.DMA,
        ],
    )
    def pos_kernel(row_hbm, col_hbm, out_hbm, col_v, row_v, buf_v, sem):
        cid = lax.axis_index("c")
        sid = lax.axis_index("s")
        wid = sid * NUM_CORES + cid
        eg = wid % n_egroups
        bg = wid // n_egroups
        i0 = eg * i_per_w
        b0 = bg * b_per_w

        pltpu.sync_copy(col_hbm.at[pl.ds(0, h)], col_v)
        pltpu.sync_copy(row_hbm.at[pl.ds(0, h)], row_v)

        def build_slab(li):
            def col_body(t, _):
                v16 = t % (LANE_TILE // LANES)
                r = t // (LANE_TILE // LANES)
                ctx = r % ct_half
                r = r // ct_half
                ws = r % SUB
                wtx = r // SUB
                vec = col_v[wtx * SUB + ws, pl.ds(ctx * LANE_TILE + v16 * LANES, LANES)]
                buf_v[li, wtx, ctx, ws, pl.ds(v16 * LANES, LANES)] = vec
                return 0

            lax.fori_loop(
                0, wt * SUB * ct_half * (LANE_TILE // LANES), col_body, 0
            )

            def row_body(t, _):
                v16 = t % (LANE_TILE // LANES)
                ctx = t // (LANE_TILE // LANES)
                vec = row_v[i0 + li, pl.ds(ctx * LANE_TILE + v16 * LANES, LANES)]

                def rep_body(r, _):
                    ws = r % SUB
                    wtx = r // SUB
                    buf_v[li, wtx, ct_half + ctx, ws, pl.ds(v16 * LANES, LANES)] = vec
                    return 0

                return lax.fori_loop(0, wt * SUB, rep_body, 0)

            lax.fori_loop(0, ct_half * (LANE_TILE // LANES), row_body, 0)

        pending = []
        for li in range(i_per_w):
            build_slab(li)
            for bb in range(b_per_w):
                cp = pltpu.async_copy(
                    buf_v.at[pl.ds(li, 1)],
                    out_hbm.at[b0 + bb, pl.ds(i0 + li, 1)],
                    sem,
                )
                pending.append(cp)
                if len(pending) >= DMA_INFLIGHT:
                    pending.pop(0).wait()
        for cp in pending:
            cp.wait()

    raw = pos_kernel(row_embed, col_embed)
    out = jnp.transpose(raw, (0, 3, 5, 1, 2, 4)).reshape(b_sz, c2, h, w)
    return out


# speedup vs baseline: 8.9142x; 1.0370x over previous
import functools

import jax
import jax.numpy as jnp
from jax import lax
from jax.experimental import pallas as pl
from jax.experimental.pallas import tpu as pltpu
from jax.experimental.pallas import tpu_sc as plsc

NUM_CORES = 2
NUM_SUBCORES = 16
LANES = 16
NUM_WORKERS = NUM_CORES * NUM_SUBCORES

SUB = 8
LANE_TILE = 128

DMA_INFLIGHT = 8


def kernel(x, row_embed, col_embed):
    b_sz = x.shape[0]
    h, w = x.shape[-2], x.shape[-1]
    n_pos, c = row_embed.shape
    c2 = 2 * c
    wt = w // SUB
    ct = c2 // LANE_TILE
    ct_half = c // LANE_TILE
    vpl = LANE_TILE // LANES

    mesh = plsc.VectorSubcoreMesh(
        core_axis_name="c",
        subcore_axis_name="s",
        num_cores=NUM_CORES,
        num_subcores=NUM_SUBCORES,
    )

    @functools.partial(
        pl.kernel,
        out_type=jax.ShapeDtypeStruct(
            (b_sz, h, wt, ct, SUB, LANE_TILE), jnp.float32
        ),
        mesh=mesh,
        scratch_types=[
            pltpu.VMEM((h, c), jnp.float32),
            pltpu.VMEM((1, c), jnp.float32),
            pltpu.VMEM((1, wt, ct, SUB, LANE_TILE), jnp.float32),
            pltpu.SemaphoreType.DMA,
        ],
    )
    def pos_kernel(row_hbm, col_hbm, out_hbm, col_v, row_v, buf_v, sem):
        cid = lax.axis_index("c")
        sid = lax.axis_index("s")
        wid = sid * NUM_CORES + cid

        pltpu.sync_copy(col_hbm.at[pl.ds(0, h)], col_v)
        pltpu.sync_copy(row_hbm.at[pl.ds(wid, 1)], row_v)

        def col_body(t, _):
            ws = t % SUB
            wtx = t // SUB
            j = wtx * SUB + ws
            for ctx in range(ct_half):
                for v in range(vpl):
                    vec = col_v[j, pl.ds(ctx * LANE_TILE + v * LANES, LANES)]
                    buf_v[0, wtx, ctx, ws, pl.ds(v * LANES, LANES)] = vec
            return 0

        lax.fori_loop(0, wt * SUB, col_body, 0)

        row_vecs = [
            row_v[0, pl.ds(ctx * LANE_TILE + v * LANES, LANES)]
            for ctx in range(ct_half)
            for v in range(vpl)
        ]

        def row_body(t, _):
            ws = t % SUB
            wtx = t // SUB
            idx = 0
            for ctx in range(ct_half):
                for v in range(vpl):
                    buf_v[0, wtx, ct_half + ctx, ws, pl.ds(v * LANES, LANES)] = (
                        row_vecs[idx]
                    )
                    idx += 1
            return 0

        lax.fori_loop(0, wt * SUB, row_body, 0)

        pending = []
        for b in range(b_sz):
            cp = pltpu.async_copy(
                buf_v, out_hbm.at[b, pl.ds(wid, 1)], sem
            )
            pending.append(cp)
            if len(pending) >= DMA_INFLIGHT:
                pending.pop(0).wait()
        for cp in pending:
            cp.wait()

    raw = pos_kernel(row_embed, col_embed)
    out = jnp.transpose(raw, (0, 3, 5, 1, 2, 4)).reshape(b_sz, c2, h, w)
    return out


# speedup vs baseline: 9.0257x; 1.0125x over previous
import functools

import jax
import jax.numpy as jnp
from jax import lax
from jax.experimental import pallas as pl
from jax.experimental.pallas import tpu as pltpu
from jax.experimental.pallas import tpu_sc as plsc

NUM_CORES = 2
NUM_SUBCORES = 16
LANES = 16
NUM_WORKERS = NUM_CORES * NUM_SUBCORES

SUB = 8
LANE_TILE = 128

DMA_INFLIGHT = 8


def kernel(x, row_embed, col_embed):
    b_sz = x.shape[0]
    h, w = x.shape[-2], x.shape[-1]
    n_pos, c = row_embed.shape
    c2 = 2 * c
    wt = w // SUB
    ct = c2 // LANE_TILE
    ct_half = c // LANE_TILE
    vpl = LANE_TILE // LANES

    mesh = plsc.VectorSubcoreMesh(
        core_axis_name="c",
        subcore_axis_name="s",
        num_cores=NUM_CORES,
        num_subcores=NUM_SUBCORES,
    )

    @functools.partial(
        pl.kernel,
        out_type=jax.ShapeDtypeStruct(
            (b_sz, h, wt, ct, SUB, LANE_TILE), jnp.float32
        ),
        mesh=mesh,
        scratch_types=[
            pltpu.VMEM((h, c), jnp.float32),
            pltpu.VMEM((1, c), jnp.float32),
            pltpu.VMEM((1, wt, ct, SUB, LANE_TILE), jnp.float32),
            pltpu.SemaphoreType.DMA,
        ],
    )
    def pos_kernel(row_hbm, col_hbm, out_hbm, col_v, row_v, buf_v, sem):
        cid = lax.axis_index("c")
        sid = lax.axis_index("s")
        wid = sid * NUM_CORES + cid

        pltpu.sync_copy(col_hbm.at[pl.ds(0, h)], col_v)
        pltpu.sync_copy(row_hbm.at[pl.ds(wid, 1)], row_v)

        def col_body(t, _):
            ws = t % SUB
            wtx = t // SUB
            j = wtx * SUB + ws
            for ctx in range(ct_half):
                for v in range(vpl):
                    vec = col_v[j, pl.ds(ctx * LANE_TILE + v * LANES, LANES)]
                    buf_v[0, wtx, ctx, ws, pl.ds(v * LANES, LANES)] = vec
            return 0

        lax.fori_loop(0, wt * SUB, col_body, 0)

        row_vecs = [
            row_v[0, pl.ds(ctx * LANE_TILE + v * LANES, LANES)]
            for ctx in range(ct_half)
            for v in range(vpl)
        ]

        def row_body(t, _):
            ws = t % SUB
            wtx = t // SUB
            idx = 0
            for ctx in range(ct_half):
                for v in range(vpl):
                    buf_v[0, wtx, ct_half + ctx, ws, pl.ds(v * LANES, LANES)] = (
                        row_vecs[idx]
                    )
                    idx += 1
            return 0

        lax.fori_loop(0, wt * SUB, row_body, 0)

        def fire(b):
            pltpu.async_copy(buf_v, out_hbm.at[b, pl.ds(wid, 1)], sem)

        def wait_one():
            pltpu.make_async_copy(buf_v, out_hbm.at[0, pl.ds(wid, 1)], sem).wait()

        def prime_body(b, _):
            fire(b)
            return 0

        lax.fori_loop(0, DMA_INFLIGHT, prime_body, 0)

        def steady_body(b, _):
            wait_one()
            fire(b)
            return 0

        lax.fori_loop(DMA_INFLIGHT, b_sz, steady_body, 0)

        def drain_body(t, _):
            wait_one()
            return 0

        lax.fori_loop(0, DMA_INFLIGHT, drain_body, 0)

    raw = pos_kernel(row_embed, col_embed)
    out = jnp.transpose(raw, (0, 3, 5, 1, 2, 4)).reshape(b_sz, c2, h, w)
    return out
